# baseline (speedup 1.0000x reference)
import functools
import math

import jax
import jax.numpy as jnp
from jax import lax
from jax.experimental import pallas as pl
from jax.experimental.pallas import tpu as pltpu
from jax.experimental.pallas import tpu_sc as plsc

_NC, _NS = 2, 16
_NW = _NC * _NS
_CHUNK = 128
_NBUF = 2


@functools.lru_cache(maxsize=None)
def _build(B, V, D, scale):
    assert B % (_NW * _CHUNK) == 0
    b_per_w = B // _NW
    n_chunks = b_per_w // _CHUNK
    lanes = D // 16
    mesh = plsc.VectorSubcoreMesh(core_axis_name="c", subcore_axis_name="s")

    @functools.partial(
        pl.kernel,
        out_type=jax.ShapeDtypeStruct((B, D), jnp.float32),
        mesh=mesh,
        compiler_params=pltpu.CompilerParams(use_tc_tiling_on_sc=False),
        scratch_types=[
            pltpu.VMEM((n_chunks, _CHUNK), jnp.int32),
            pltpu.VMEM((_CHUNK, D), jnp.float32),
            pltpu.VMEM((_CHUNK, D), jnp.float32),
            pltpu.VMEM((_CHUNK, D), jnp.float32),
            pltpu.VMEM((_CHUNK, D), jnp.float32),
            pltpu.SemaphoreType.DMA,
            pltpu.SemaphoreType.DMA,
            pltpu.SemaphoreType.DMA,
            pltpu.SemaphoreType.DMA,
        ],
    )
    def kern(idx_hbm, table_hbm, out_hbm, idx_v, in0, in1, ot0, ot1,
             g0, g1, s0, s1):
        ins, outs = (in0, in1), (ot0, ot1)
        gsem, ssem = (g0, g1), (s0, s1)
        wid = lax.axis_index("s") * _NC + lax.axis_index("c")
        base = wid * b_per_w

        pltpu.sync_copy(idx_hbm.at[pl.ds(wid * n_chunks, n_chunks)], idx_v)

        def gather(g, b):
            return pltpu.make_async_copy(
                table_hbm.at[idx_v.at[g]], ins[b], gsem[b])

        def scatter(g, b):
            return pltpu.make_async_copy(
                outs[b], out_hbm.at[pl.ds(base + g * _CHUNK, _CHUNK)], ssem[b])

        for b in range(_NBUF):
            gather(b, b).start()

        @pl.loop(0, n_chunks, step=_NBUF)
        def _(c0):
            for b in range(_NBUF):
                g = c0 + b
                gather(g, b).wait()

                @pl.when(g >= _NBUF)
                def _():
                    scatter(g - _NBUF, b).wait()

                @plsc.parallel_loop(0, _CHUNK, unroll=4)
                def _(r):
                    for c in range(lanes):
                        outs[b][r, pl.ds(c * 16, 16)] = (
                            ins[b][r, pl.ds(c * 16, 16)] * scale)

                scatter(g, b).start()

                @pl.when(g + _NBUF < n_chunks)
                def _():
                    gather(g + _NBUF, b).start()

        for b in range(_NBUF):
            scatter(n_chunks - _NBUF + b, b).wait()

    return kern


def kernel(input_ids, embeddings):
    batch, hist = input_ids.shape
    V, D = embeddings.shape
    B = batch * hist
    scale = float(math.sqrt(D))
    idx = input_ids.reshape(B // _CHUNK, _CHUNK).astype(jnp.int32)
    out = _build(B, V, D, scale)(idx, embeddings)
    return out.reshape(batch, hist, D)


# speedup vs baseline: 1.0199x; 1.0199x over previous
import functools
import math

import jax
import jax.numpy as jnp
from jax import lax
from jax.experimental import pallas as pl
from jax.experimental.pallas import tpu as pltpu
from jax.experimental.pallas import tpu_sc as plsc

_NC, _NS = 2, 16
_NW = _NC * _NS
_NBUF = 2


@functools.lru_cache(maxsize=None)
def _build(batch, hist, V, D, scale):
    assert batch % _NW == 0 and D % 16 == 0
    rows_w = batch // _NW
    lanes = D // 16
    segs = []
    off = 0
    while off < hist:
        n = min(128, hist - off)
        segs.append((off, n))
        off += n
    mesh = plsc.VectorSubcoreMesh(core_axis_name="c", subcore_axis_name="s")

    @functools.partial(
        pl.kernel,
        out_type=jax.ShapeDtypeStruct((batch, hist, D), jnp.float32),
        mesh=mesh,
        compiler_params=pltpu.CompilerParams(use_tc_tiling_on_sc=False),
        scratch_types=[
            pltpu.VMEM((rows_w, hist), jnp.int32),
            pltpu.VMEM((hist, D), jnp.float32),
            pltpu.VMEM((hist, D), jnp.float32),
            pltpu.VMEM((hist, D), jnp.float32),
            pltpu.VMEM((hist, D), jnp.float32),
            pltpu.SemaphoreType.DMA,
            pltpu.SemaphoreType.DMA,
            pltpu.SemaphoreType.DMA,
            pltpu.SemaphoreType.DMA,
        ],
    )
    def kern(idx_hbm, table_hbm, out_hbm, idx_v, in0, in1, ot0, ot1,
             g0, g1, s0, s1):
        ins, outs = (in0, in1), (ot0, ot1)
        gsem, ssem = (g0, g1), (s0, s1)
        wid = lax.axis_index("s") * _NC + lax.axis_index("c")
        base = wid * rows_w

        pltpu.sync_copy(idx_hbm.at[pl.ds(base, rows_w)], idx_v)

        def gathers(r, b):
            return [
                pltpu.make_async_copy(
                    table_hbm.at[idx_v.at[r, pl.ds(o, n)]],
                    ins[b].at[pl.ds(o, n)], gsem[b])
                for o, n in segs
            ]

        def scatter(r, b):
            return pltpu.make_async_copy(outs[b], out_hbm.at[base + r],
                                         ssem[b])

        for b in range(_NBUF):
            for c in gathers(b, b):
                c.start()

        @pl.loop(0, rows_w, step=_NBUF)
        def _(r0):
            for b in range(_NBUF):
                r = r0 + b
                for c in gathers(r, b):
                    c.wait()

                @pl.when(r >= _NBUF)
                def _():
                    scatter(r - _NBUF, b).wait()

                @plsc.parallel_loop(0, hist, unroll=4)
                def _(i):
                    for c in range(lanes):
                        outs[b][i, pl.ds(c * 16, 16)] = (
                            ins[b][i, pl.ds(c * 16, 16)] * scale)

                scatter(r, b).start()

                @pl.when(r + _NBUF < rows_w)
                def _():
                    for c in gathers(r + _NBUF, b):
                        c.start()

        for b in range(_NBUF):
            scatter(rows_w - _NBUF + b, b).wait()

    return kern


def kernel(input_ids, embeddings):
    batch, hist = input_ids.shape
    V, D = embeddings.shape
    scale = float(math.sqrt(D))
    idx = input_ids.astype(jnp.int32)
    return _build(batch, hist, V, D, scale)(idx, embeddings)
